# baseline (speedup 1.0000x reference)
import functools

import jax
import jax.numpy as jnp
from jax import lax
from jax.experimental import pallas as pl
from jax.experimental.pallas import tpu as pltpu
from jax.experimental.pallas import tpu_sc as plsc

_NC = 2
_NS = 16
_NW = _NC * _NS
_CHUNK = 128
_NBUF = 8
_LOOKAHEAD = 4

_SPLIT = 500224


def _table_transpose(wt):
  V = wt.shape[1]
  nj = _SPLIT // 512

  def body(lo_ref, hi_ref, out_ref):
    out_ref[:, 0:64] = lo_ref[...].T
    out_ref[:, 64:128] = hi_ref[...].T

  return pl.pallas_call(
      body,
      grid=(nj,),
      in_specs=[
          pl.BlockSpec((64, 512), lambda j: (0, j)),
          pl.BlockSpec((64, 512), lambda j: (0, j + nj)),
      ],
      out_specs=pl.BlockSpec((512, 128), lambda j: (j, 0)),
      out_shape=jax.ShapeDtypeStruct((_SPLIT, 128), jnp.float32),
  )(wt, wt)


def _out_transpose(in3):
  T, P, _ = in3.shape
  nc = P // 512

  def body(in_ref, out_ref):
    blk = in_ref[0]
    out_ref[0, :, 0:512] = blk[:, 0:64].T
    out_ref[0, :, 512:1024] = blk[:, 64:128].T

  return pl.pallas_call(
      body,
      grid=(T, nc),
      in_specs=[pl.BlockSpec((1, 512, 128), lambda t, c: (t, c, 0))],
      out_specs=pl.BlockSpec((1, 64, 1024), lambda t, c: (t, 0, c)),
      out_shape=jax.ShapeDtypeStruct((T, 64, 2 * P), jnp.float32),
  )(in3)


def _make_gather(Vp: int, B: int, D: int, n_chunks: int):
  mesh = plsc.VectorSubcoreMesh(core_axis_name="c", subcore_axis_name="s")

  @functools.partial(
      pl.kernel,
      out_type=jax.ShapeDtypeStruct((B, D), jnp.float32),
      mesh=mesh,
      scratch_types=[
          pltpu.VMEM((n_chunks, _CHUNK), jnp.int32),
          pltpu.VMEM((_NBUF, _CHUNK, D), jnp.float32),
          pltpu.SemaphoreType.DMA,
          pltpu.SemaphoreType.DMA,
      ],
      compiler_params=pltpu.CompilerParams(use_tc_tiling_on_sc=False),
  )
  def gather_kernel(table_hbm, idx_hbm, out_hbm, idx_v, rows, gsem, wsem):
    wid = lax.axis_index("s") * _NC + lax.axis_index("c")
    base = wid * (n_chunks * _CHUNK)
    pltpu.sync_copy(idx_hbm.at[wid], idx_v)

    def g(j, b):
      pltpu.async_copy(table_hbm.at[idx_v.at[j]], rows.at[b], gsem)

    def wg(b):
      pltpu.make_async_copy(
          table_hbm.at[idx_v.at[0]], rows.at[b], gsem).wait()

    def w(j, b):
      pltpu.async_copy(
          rows.at[b], out_hbm.at[pl.ds(base + j * _CHUNK, _CHUNK)], wsem)

    def ww(b):
      pltpu.make_async_copy(
          rows.at[b], out_hbm.at[pl.ds(base, _CHUNK)], wsem).wait()

    LA, NB = _LOOKAHEAD, _NBUF
    n_groups = n_chunks // NB

    for b in range(LA):
      g(b, b)

    for b in range(NB):
      wg(b)
      w(b, b)
      bn = (b + LA) % NB
      if b >= LA:
        ww(bn)
      g(b + LA, bn)

    def body(k, carry):
      j0 = k * NB
      for b in range(NB):
        wg(b)
        w(j0 + b, b)
        bn = (b + LA) % NB
        ww(bn)
        g(j0 + b + LA, bn)
      return carry

    lax.fori_loop(1, n_groups - 1, body, 0)

    j0 = (n_groups - 1) * NB
    for b in range(NB):
      wg(b)
      w(j0 + b, b)
      if b < NB - LA:
        bn = (b + LA) % NB
        ww(bn)
        g(j0 + b + LA, bn)

    for b in range(NB):
      ww(b)

  return gather_kernel


def kernel(x, weight):
  BATCH, HIST = x.shape
  V, D = weight.shape
  B = BATCH * HIST
  assert B % (_NW * _CHUNK) == 0
  n_chunks = B // (_NW * _CHUNK)
  Vp = 2 * _SPLIT

  table2 = _table_transpose(weight.T)
  table_lin = table2.reshape(Vp, D)

  half = BATCH // 2
  x3 = (x.T.reshape(HIST, half // 512, 2, 512)
        .transpose(0, 1, 3, 2).reshape(HIST, half, 2))
  v = x3.astype(jnp.int32)
  vp = jnp.where(v < _SPLIT, 2 * v, 2 * (v - _SPLIT) + 1)
  idx = vp.reshape(_NW, n_chunks, _CHUNK)

  out_tr = _make_gather(Vp, B, D, n_chunks)(table_lin, idx)

  in3 = out_tr.reshape(HIST, half, 2 * D)
  final = _out_transpose(in3)
  return final.transpose(2, 0, 1)


# speedup vs baseline: 1.0621x; 1.0621x over previous
import functools

import jax
import jax.numpy as jnp
from jax import lax
from jax.experimental import pallas as pl
from jax.experimental.pallas import tpu as pltpu
from jax.experimental.pallas import tpu_sc as plsc

_NC = 2
_NS = 16
_NW = _NC * _NS
_CHUNK = 128
_NBUF = 8
_LOOKAHEAD = 4

_SPLIT = 500224


def _table_transpose(wt):
  V = wt.shape[1]
  nj = _SPLIT // 512

  def body(lo_ref, hi_ref, out_ref):
    out_ref[...] = jnp.concatenate([lo_ref[...], hi_ref[...]], axis=0).T

  return pl.pallas_call(
      body,
      grid=(nj,),
      in_specs=[
          pl.BlockSpec((64, 512), lambda j: (0, j)),
          pl.BlockSpec((64, 512), lambda j: (0, j + nj)),
      ],
      out_specs=pl.BlockSpec((512, 128), lambda j: (j, 0)),
      out_shape=jax.ShapeDtypeStruct((_SPLIT, 128), jnp.float32),
  )(wt, wt)


def _out_transpose(in3):
  T, P, _ = in3.shape
  nc = P // 512

  def body(in_ref, out_ref):
    t = in_ref[0].T
    out_ref[0, :, 0:512] = t[0:64, :]
    out_ref[0, :, 512:1024] = t[64:128, :]

  return pl.pallas_call(
      body,
      grid=(T, nc),
      in_specs=[pl.BlockSpec((1, 512, 128), lambda t, c: (t, c, 0))],
      out_specs=pl.BlockSpec((1, 64, 1024), lambda t, c: (t, 0, c)),
      out_shape=jax.ShapeDtypeStruct((T, 64, 2 * P), jnp.float32),
  )(in3)


def _make_gather(Vp: int, B: int, D: int, n_chunks: int):
  mesh = plsc.VectorSubcoreMesh(core_axis_name="c", subcore_axis_name="s")

  @functools.partial(
      pl.kernel,
      out_type=jax.ShapeDtypeStruct((B, D), jnp.float32),
      mesh=mesh,
      scratch_types=[
          pltpu.VMEM((n_chunks, _CHUNK), jnp.int32),
          pltpu.VMEM((_NBUF, _CHUNK, D), jnp.float32),
          pltpu.SemaphoreType.DMA,
          pltpu.SemaphoreType.DMA,
      ],
      compiler_params=pltpu.CompilerParams(use_tc_tiling_on_sc=False),
  )
  def gather_kernel(table_hbm, idx_hbm, out_hbm, idx_v, rows, gsem, wsem):
    wid = lax.axis_index("s") * _NC + lax.axis_index("c")
    base = wid * (n_chunks * _CHUNK)
    pltpu.sync_copy(idx_hbm.at[wid], idx_v)

    def g(j, b):
      pltpu.async_copy(table_hbm.at[idx_v.at[j]], rows.at[b], gsem)

    def wg(b):
      pltpu.make_async_copy(
          table_hbm.at[idx_v.at[0]], rows.at[b], gsem).wait()

    def w(j, b):
      pltpu.async_copy(
          rows.at[b], out_hbm.at[pl.ds(base + j * _CHUNK, _CHUNK)], wsem)

    def ww(b):
      pltpu.make_async_copy(
          rows.at[b], out_hbm.at[pl.ds(base, _CHUNK)], wsem).wait()

    LA, NB = _LOOKAHEAD, _NBUF
    n_groups = n_chunks // NB

    for b in range(LA):
      g(b, b)

    for b in range(NB):
      wg(b)
      w(b, b)
      bn = (b + LA) % NB
      if b >= LA:
        ww(bn)
      g(b + LA, bn)

    def body(k, carry):
      j0 = k * NB
      for b in range(NB):
        wg(b)
        w(j0 + b, b)
        bn = (b + LA) % NB
        ww(bn)
        g(j0 + b + LA, bn)
      return carry

    lax.fori_loop(1, n_groups - 1, body, 0)

    j0 = (n_groups - 1) * NB
    for b in range(NB):
      wg(b)
      w(j0 + b, b)
      if b < NB - LA:
        bn = (b + LA) % NB
        ww(bn)
        g(j0 + b + LA, bn)

    for b in range(NB):
      ww(b)

  return gather_kernel


def kernel(x, weight):
  BATCH, HIST = x.shape
  V, D = weight.shape
  B = BATCH * HIST
  assert B % (_NW * _CHUNK) == 0
  n_chunks = B // (_NW * _CHUNK)
  Vp = 2 * _SPLIT

  table2 = _table_transpose(weight.T)
  table_lin = table2.reshape(Vp, D)

  half = BATCH // 2
  x3 = (x.T.reshape(HIST, half // 512, 2, 512)
        .transpose(0, 1, 3, 2).reshape(HIST, half, 2))
  v = x3.astype(jnp.int32)
  vp = jnp.where(v < _SPLIT, 2 * v, 2 * (v - _SPLIT) + 1)
  idx = vp.reshape(_NW, n_chunks, _CHUNK)

  out_tr = _make_gather(Vp, B, D, n_chunks)(table_lin, idx)

  in3 = out_tr.reshape(HIST, half, 2 * D)
  final = _out_transpose(in3)
  return final.transpose(2, 0, 1)


# speedup vs baseline: 2.0178x; 1.8998x over previous
import functools

import jax
import jax.numpy as jnp
from jax import lax
from jax.experimental import pallas as pl
from jax.experimental.pallas import tpu as pltpu
from jax.experimental.pallas import tpu_sc as plsc

_NC = 2
_NS = 16
_NW = _NC * _NS
_CHUNK = 128
_NBUF = 8
_LOOKAHEAD = 4

_SPLIT = 507904


def _table_transpose(wt):
  V = wt.shape[1]
  C = 4096
  nj = _SPLIT // C
  last = (V - 1) // C

  def body(lo_ref, hi_ref, out_ref):
    out_ref[...] = jnp.concatenate([lo_ref[...], hi_ref[...]], axis=0).T

  return pl.pallas_call(
      body,
      grid=(nj,),
      in_specs=[
          pl.BlockSpec((64, C), lambda j: (0, j)),
          pl.BlockSpec((64, C), lambda j: (0, jnp.minimum(j + nj, last))),
      ],
      out_specs=pl.BlockSpec((C, 128), lambda j: (j, 0)),
      out_shape=jax.ShapeDtypeStruct((_SPLIT, 128), jnp.float32),
  )(wt, wt)


def _out_transpose(in3):
  T, P, _ = in3.shape
  Q = 2048
  nc = P // Q

  def body(in_ref, out_ref):
    t = in_ref[0].T
    for k in range(Q // 512):
      out_ref[0, :, 1024 * k:1024 * k + 512] = t[0:64, 512 * k:512 * k + 512]
      out_ref[0, :, 1024 * k + 512:1024 * k + 1024] = (
          t[64:128, 512 * k:512 * k + 512])

  return pl.pallas_call(
      body,
      grid=(T, nc),
      in_specs=[pl.BlockSpec((1, Q, 128), lambda t, c: (t, c, 0))],
      out_specs=pl.BlockSpec((1, 64, 2 * Q), lambda t, c: (t, 0, c)),
      out_shape=jax.ShapeDtypeStruct((T, 64, 2 * P), jnp.float32),
  )(in3)


def _make_gather(Vp: int, B: int, D: int, n_chunks: int):
  mesh = plsc.VectorSubcoreMesh(core_axis_name="c", subcore_axis_name="s")

  @functools.partial(
      pl.kernel,
      out_type=jax.ShapeDtypeStruct((B, D), jnp.float32),
      mesh=mesh,
      scratch_types=[
          pltpu.VMEM((n_chunks, _CHUNK), jnp.int32),
          pltpu.VMEM((_NBUF, _CHUNK, D), jnp.float32),
          pltpu.SemaphoreType.DMA,
          pltpu.SemaphoreType.DMA,
      ],
      compiler_params=pltpu.CompilerParams(use_tc_tiling_on_sc=False),
  )
  def gather_kernel(table_hbm, idx_hbm, out_hbm, idx_v, rows, gsem, wsem):
    wid = lax.axis_index("s") * _NC + lax.axis_index("c")
    base = wid * (n_chunks * _CHUNK)
    pltpu.sync_copy(idx_hbm.at[wid], idx_v)

    def g(j, b):
      pltpu.async_copy(table_hbm.at[idx_v.at[j]], rows.at[b], gsem)

    def wg(b):
      pltpu.make_async_copy(
          table_hbm.at[idx_v.at[0]], rows.at[b], gsem).wait()

    def w(j, b):
      pltpu.async_copy(
          rows.at[b], out_hbm.at[pl.ds(base + j * _CHUNK, _CHUNK)], wsem)

    def ww(b):
      pltpu.make_async_copy(
          rows.at[b], out_hbm.at[pl.ds(base, _CHUNK)], wsem).wait()

    LA, NB = _LOOKAHEAD, _NBUF
    n_groups = n_chunks // NB

    for b in range(LA):
      g(b, b)

    for b in range(NB):
      wg(b)
      w(b, b)
      bn = (b + LA) % NB
      if b >= LA:
        ww(bn)
      g(b + LA, bn)

    def body(k, carry):
      j0 = k * NB
      for b in range(NB):
        wg(b)
        w(j0 + b, b)
        bn = (b + LA) % NB
        ww(bn)
        g(j0 + b + LA, bn)
      return carry

    lax.fori_loop(1, n_groups - 1, body, 0)

    j0 = (n_groups - 1) * NB
    for b in range(NB):
      wg(b)
      w(j0 + b, b)
      if b < NB - LA:
        bn = (b + LA) % NB
        ww(bn)
        g(j0 + b + LA, bn)

    for b in range(NB):
      ww(b)

  return gather_kernel


def kernel(x, weight):
  BATCH, HIST = x.shape
  V, D = weight.shape
  B = BATCH * HIST
  assert B % (_NW * _CHUNK) == 0
  n_chunks = B // (_NW * _CHUNK)
  Vp = 2 * _SPLIT

  table2 = _table_transpose(weight.T)
  table_lin = table2.reshape(Vp, D)

  half = BATCH // 2
  x3 = (x.T.reshape(HIST, half // 512, 2, 512)
        .transpose(0, 1, 3, 2).reshape(HIST, half, 2))
  v = x3.astype(jnp.int32)
  vp = jnp.where(v < _SPLIT, 2 * v, 2 * (v - _SPLIT) + 1)
  idx = vp.reshape(_NW, n_chunks, _CHUNK)

  out_tr = _make_gather(Vp, B, D, n_chunks)(table_lin, idx)

  in3 = out_tr.reshape(HIST, half, 2 * D)
  final = _out_transpose(in3)
  return final.transpose(2, 0, 1)


# speedup vs baseline: 2.2677x; 1.1238x over previous
import functools

import jax
import jax.numpy as jnp
from jax import lax
from jax.experimental import pallas as pl
from jax.experimental.pallas import tpu as pltpu
from jax.experimental.pallas import tpu_sc as plsc

_NC = 2
_NS = 16
_NW = _NC * _NS
_CHUNK = 128
_NBUF = 8
_LOOKAHEAD = 4

_SPLIT = 507904


def _table_transpose(wt):
  V = wt.shape[1]
  C = 8192
  nj = _SPLIT // C
  last = (V - 1) // C

  def body(lo_ref, hi_ref, out_ref):
    out_ref[...] = jnp.concatenate([lo_ref[...], hi_ref[...]], axis=0).T

  return pl.pallas_call(
      body,
      grid=(nj,),
      in_specs=[
          pl.BlockSpec((64, C), lambda j: (0, j)),
          pl.BlockSpec((64, C), lambda j: (0, jnp.minimum(j + nj, last))),
      ],
      out_specs=pl.BlockSpec((C, 128), lambda j: (j, 0)),
      out_shape=jax.ShapeDtypeStruct((_SPLIT, 128), jnp.float32),
  )(wt, wt)


def _out_transpose(in3):
  T, P, _ = in3.shape
  Q = 4096
  nc = P // Q

  def body(in_ref, out_ref):
    t = in_ref[0].T
    for k in range(Q // 512):
      out_ref[0, :, 1024 * k:1024 * k + 512] = t[0:64, 512 * k:512 * k + 512]
      out_ref[0, :, 1024 * k + 512:1024 * k + 1024] = (
          t[64:128, 512 * k:512 * k + 512])

  return pl.pallas_call(
      body,
      grid=(T, nc),
      in_specs=[pl.BlockSpec((1, Q, 128), lambda t, c: (t, c, 0))],
      out_specs=pl.BlockSpec((1, 64, 2 * Q), lambda t, c: (t, 0, c)),
      out_shape=jax.ShapeDtypeStruct((T, 64, 2 * P), jnp.float32),
  )(in3)


def _make_gather(Vp: int, B: int, D: int, n_chunks: int):
  mesh = plsc.VectorSubcoreMesh(core_axis_name="c", subcore_axis_name="s")

  @functools.partial(
      pl.kernel,
      out_type=jax.ShapeDtypeStruct((B, D), jnp.float32),
      mesh=mesh,
      scratch_types=[
          pltpu.VMEM((n_chunks, _CHUNK), jnp.int32),
          pltpu.VMEM((_NBUF, _CHUNK, D), jnp.float32),
          pltpu.SemaphoreType.DMA,
          pltpu.SemaphoreType.DMA,
      ],
      compiler_params=pltpu.CompilerParams(use_tc_tiling_on_sc=False),
  )
  def gather_kernel(table_hbm, idx_hbm, out_hbm, idx_v, rows, gsem, wsem):
    wid = lax.axis_index("s") * _NC + lax.axis_index("c")
    base = wid * (n_chunks * _CHUNK)
    pltpu.sync_copy(idx_hbm.at[wid], idx_v)

    def g(j, b):
      pltpu.async_copy(table_hbm.at[idx_v.at[j]], rows.at[b], gsem)

    def wg(b):
      pltpu.make_async_copy(
          table_hbm.at[idx_v.at[0]], rows.at[b], gsem).wait()

    def w(j, b):
      pltpu.async_copy(
          rows.at[b], out_hbm.at[pl.ds(base + j * _CHUNK, _CHUNK)], wsem)

    def ww(b):
      pltpu.make_async_copy(
          rows.at[b], out_hbm.at[pl.ds(base, _CHUNK)], wsem).wait()

    LA, NB = _LOOKAHEAD, _NBUF
    n_groups = n_chunks // NB

    for b in range(LA):
      g(b, b)

    for b in range(NB):
      wg(b)
      w(b, b)
      bn = (b + LA) % NB
      if b >= LA:
        ww(bn)
      g(b + LA, bn)

    def body(k, carry):
      j0 = k * NB
      for b in range(NB):
        wg(b)
        w(j0 + b, b)
        bn = (b + LA) % NB
        ww(bn)
        g(j0 + b + LA, bn)
      return carry

    lax.fori_loop(1, n_groups - 1, body, 0)

    j0 = (n_groups - 1) * NB
    for b in range(NB):
      wg(b)
      w(j0 + b, b)
      if b < NB - LA:
        bn = (b + LA) % NB
        ww(bn)
        g(j0 + b + LA, bn)

    for b in range(NB):
      ww(b)

  return gather_kernel


def kernel(x, weight):
  BATCH, HIST = x.shape
  V, D = weight.shape
  B = BATCH * HIST
  assert B % (_NW * _CHUNK) == 0
  n_chunks = B // (_NW * _CHUNK)
  Vp = 2 * _SPLIT

  table2 = _table_transpose(weight.T)
  table_lin = table2.reshape(Vp, D)

  half = BATCH // 2
  x3 = (x.T.reshape(HIST, half // 512, 2, 512)
        .transpose(0, 1, 3, 2).reshape(HIST, half, 2))
  v = x3.astype(jnp.int32)
  vp = jnp.where(v < _SPLIT, 2 * v, 2 * (v - _SPLIT) + 1)
  idx = vp.reshape(_NW, n_chunks, _CHUNK)

  out_tr = _make_gather(Vp, B, D, n_chunks)(table_lin, idx)

  in3 = out_tr.reshape(HIST, half, 2 * D)
  final = _out_transpose(in3)
  return final.transpose(2, 0, 1)


# speedup vs baseline: 2.3653x; 1.0431x over previous
import functools

import jax
import jax.numpy as jnp
from jax import lax
from jax.experimental import pallas as pl
from jax.experimental.pallas import tpu as pltpu
from jax.experimental.pallas import tpu_sc as plsc

_NC = 2
_NS = 16
_NW = _NC * _NS
_CHUNK = 128
_NBUF = 8
_LOOKAHEAD = 4

_SPLIT = 507904


def _table_transpose(wt):
  V = wt.shape[1]
  C = 16384
  nj = _SPLIT // C
  last = (V - 1) // C

  def body(lo_ref, hi_ref, out_ref):
    out_ref[...] = jnp.concatenate([lo_ref[...], hi_ref[...]], axis=0).T

  return pl.pallas_call(
      body,
      grid=(nj,),
      in_specs=[
          pl.BlockSpec((64, C), lambda j: (0, j)),
          pl.BlockSpec((64, C), lambda j: (0, jnp.minimum(j + nj, last))),
      ],
      out_specs=pl.BlockSpec((C, 128), lambda j: (j, 0)),
      out_shape=jax.ShapeDtypeStruct((_SPLIT, 128), jnp.float32),
  )(wt, wt)


def _out_transpose(in3):
  T, P, _ = in3.shape
  Q = 8192
  nc = P // Q

  def body(in_ref, out_ref):
    t = in_ref[0].T
    for k in range(Q // 512):
      out_ref[0, :, 1024 * k:1024 * k + 512] = t[0:64, 512 * k:512 * k + 512]
      out_ref[0, :, 1024 * k + 512:1024 * k + 1024] = (
          t[64:128, 512 * k:512 * k + 512])

  return pl.pallas_call(
      body,
      grid=(T, nc),
      in_specs=[pl.BlockSpec((1, Q, 128), lambda t, c: (t, c, 0))],
      out_specs=pl.BlockSpec((1, 64, 2 * Q), lambda t, c: (t, 0, c)),
      out_shape=jax.ShapeDtypeStruct((T, 64, 2 * P), jnp.float32),
  )(in3)


def _make_gather(Vp: int, B: int, D: int, n_chunks: int):
  mesh = plsc.VectorSubcoreMesh(core_axis_name="c", subcore_axis_name="s")

  @functools.partial(
      pl.kernel,
      out_type=jax.ShapeDtypeStruct((B, D), jnp.float32),
      mesh=mesh,
      scratch_types=[
          pltpu.VMEM((n_chunks, _CHUNK), jnp.int32),
          pltpu.VMEM((_NBUF, _CHUNK, D), jnp.float32),
          pltpu.SemaphoreType.DMA,
          pltpu.SemaphoreType.DMA,
      ],
      compiler_params=pltpu.CompilerParams(use_tc_tiling_on_sc=False),
  )
  def gather_kernel(table_hbm, idx_hbm, out_hbm, idx_v, rows, gsem, wsem):
    wid = lax.axis_index("s") * _NC + lax.axis_index("c")
    base = wid * (n_chunks * _CHUNK)
    pltpu.sync_copy(idx_hbm.at[wid], idx_v)

    def g(j, b):
      pltpu.async_copy(table_hbm.at[idx_v.at[j]], rows.at[b], gsem)

    def wg(b):
      pltpu.make_async_copy(
          table_hbm.at[idx_v.at[0]], rows.at[b], gsem).wait()

    def w(j, b):
      pltpu.async_copy(
          rows.at[b], out_hbm.at[pl.ds(base + j * _CHUNK, _CHUNK)], wsem)

    def ww(b):
      pltpu.make_async_copy(
          rows.at[b], out_hbm.at[pl.ds(base, _CHUNK)], wsem).wait()

    LA, NB = _LOOKAHEAD, _NBUF
    n_groups = n_chunks // NB

    for b in range(LA):
      g(b, b)

    for b in range(NB):
      wg(b)
      w(b, b)
      bn = (b + LA) % NB
      if b >= LA:
        ww(bn)
      g(b + LA, bn)

    def body(k, carry):
      j0 = k * NB
      for b in range(NB):
        wg(b)
        w(j0 + b, b)
        bn = (b + LA) % NB
        ww(bn)
        g(j0 + b + LA, bn)
      return carry

    lax.fori_loop(1, n_groups - 1, body, 0)

    j0 = (n_groups - 1) * NB
    for b in range(NB):
      wg(b)
      w(j0 + b, b)
      if b < NB - LA:
        bn = (b + LA) % NB
        ww(bn)
        g(j0 + b + LA, bn)

    for b in range(NB):
      ww(b)

  return gather_kernel


def kernel(x, weight):
  BATCH, HIST = x.shape
  V, D = weight.shape
  B = BATCH * HIST
  assert B % (_NW * _CHUNK) == 0
  n_chunks = B // (_NW * _CHUNK)
  Vp = 2 * _SPLIT

  table2 = _table_transpose(weight.T)
  table_lin = table2.reshape(Vp, D)

  half = BATCH // 2
  x3 = (x.T.reshape(HIST, half // 512, 2, 512)
        .transpose(0, 1, 3, 2).reshape(HIST, half, 2))
  v = x3.astype(jnp.int32)
  vp = jnp.where(v < _SPLIT, 2 * v, 2 * (v - _SPLIT) + 1)
  idx = vp.reshape(_NW, n_chunks, _CHUNK)

  out_tr = _make_gather(Vp, B, D, n_chunks)(table_lin, idx)

  in3 = out_tr.reshape(HIST, half, 2 * D)
  final = _out_transpose(in3)
  return final.transpose(2, 0, 1)
